# Initial kernel scaffold; baseline (speedup 1.0000x reference)
import jax
import jax.numpy as jnp
from jax.experimental import pallas as pl


def kernel(x, coord, edge_attr, edge_index, pe_w1, pe_b1, pe_w2, pe_b2, pe_p_w1, pe_p_b1, pe_p_w2, pe_p_b2, ni_w, ni_b, mm_w1, mm_b1, mm_w2, mm_b2, geo_w, nm_w1, nm_b1, nm_w2, nm_b2, cm_w1, cm_b1, cm_w2, frequencies):
    raise NotImplementedError("write your pallas kernel here")


# speedup vs baseline: 21.3826x; 21.3826x over previous
import jax
import jax.numpy as jnp
import numpy as np
from jax import lax
from jax.experimental import pallas as pl
from jax.experimental.pallas import tpu as pltpu
from jax.experimental.pallas import tpu_sc as plsc

N = 10000
E = 160000
D = 128
NFB = 32

_NCORES = 2
_NSUB = 16
_NTILES = _NCORES * _NSUB
_CH = 128
_CPT = 40
_EP = _NTILES * _CPT * _CH
_NP = 10240
_RPS = _NP // _NSUB

_BE1 = 4096
_BE2 = 2048
_BN = 2048

_f32 = jnp.float32

_SC_LINEAR = pltpu.CompilerParams(use_tc_tiling_on_sc=False)


def _silu(v):
    return v * jax.nn.sigmoid(v)


def _mesh():
    return plsc.VectorSubcoreMesh(core_axis_name="c", subcore_axis_name="s")


def _sc_gather_x(xp, row2d, col2d):
    def body(xp_h, row_h, col_h, xs_h, xt_h,
             ridx, cidx, xsbuf, xtbuf, gsem, ssem):
        c = lax.axis_index("c")
        s = lax.axis_index("s")
        wid = s * _NCORES + c
        pltpu.sync_copy(row_h.at[pl.ds(wid * _CPT, _CPT)], ridx)
        pltpu.sync_copy(col_h.at[pl.ds(wid * _CPT, _CPT)], cidx)

        def step(j, carry):
            base = (wid * _CPT + j) * _CH
            d1 = pltpu.async_copy(xp_h.at[ridx.at[j]], xsbuf, gsem)
            d2 = pltpu.async_copy(xp_h.at[cidx.at[j]], xtbuf, gsem)
            d1.wait(); d2.wait()
            o1 = pltpu.async_copy(xsbuf, xs_h.at[pl.ds(base, _CH)], ssem)
            o2 = pltpu.async_copy(xtbuf, xt_h.at[pl.ds(base, _CH)], ssem)
            o1.wait(); o2.wait()
            return carry

        lax.fori_loop(0, _CPT, step, 0)

    f = pl.kernel(
        body,
        out_type=[
            jax.ShapeDtypeStruct((_EP, D), _f32),
            jax.ShapeDtypeStruct((_EP, D), _f32),
        ],
        mesh=_mesh(),
        scratch_types=[
            pltpu.VMEM((_CPT, _CH), jnp.int32),
            pltpu.VMEM((_CPT, _CH), jnp.int32),
            pltpu.VMEM((_CH, D), _f32),
            pltpu.VMEM((_CH, D), _f32),
            pltpu.SemaphoreType.DMA,
            pltpu.SemaphoreType.DMA,
        ],
    )
    return f(xp, row2d, col2d)


def _sc_gather_coord(coordp, row2d, col2d):
    def body(cp_h, row_h, col_h, cr_h, cc_h,
             ridx, cidx, crbuf, ccbuf, gsem, ssem):
        c = lax.axis_index("c")
        s = lax.axis_index("s")
        wid = s * _NCORES + c
        pltpu.sync_copy(row_h.at[pl.ds(wid * _CPT, _CPT)], ridx)
        pltpu.sync_copy(col_h.at[pl.ds(wid * _CPT, _CPT)], cidx)

        def step(j, carry):
            base = (wid * _CPT + j) * _CH
            d1 = pltpu.async_copy(cp_h.at[ridx.at[j]], crbuf, gsem)
            d2 = pltpu.async_copy(cp_h.at[cidx.at[j]], ccbuf, gsem)
            d1.wait(); d2.wait()
            o1 = pltpu.async_copy(crbuf, cr_h.at[pl.ds(base, _CH)], ssem)
            o2 = pltpu.async_copy(ccbuf, cc_h.at[pl.ds(base, _CH)], ssem)
            o1.wait(); o2.wait()
            return carry

        lax.fori_loop(0, _CPT, step, 0)

    f = pl.kernel(
        body,
        out_type=[
            jax.ShapeDtypeStruct((_EP, 16), _f32),
            jax.ShapeDtypeStruct((_EP, 16), _f32),
        ],
        mesh=_mesh(),
        scratch_types=[
            pltpu.VMEM((_CPT, _CH), jnp.int32),
            pltpu.VMEM((_CPT, _CH), jnp.int32),
            pltpu.VMEM((_CH, 16), _f32),
            pltpu.VMEM((_CH, 16), _f32),
            pltpu.SemaphoreType.DMA,
            pltpu.SemaphoreType.DMA,
        ],
        compiler_params=_SC_LINEAR,
    )
    return f(coordp, row2d, col2d)


def _sc_scatter_m(m, row2d):
    def body(m_h, row_h, pm0_h, pm1_h,
             ridx, mbuf, sh_m, lsem, asem):
        c = lax.axis_index("c")
        s = lax.axis_index("s")
        wid = s * _NCORES + c

        def zrow(i, carry):
            def zcol(k, carry2):
                mbuf[i, pl.ds(k * 16, 16)] = jnp.zeros((16,), _f32)
                return carry2
            return lax.fori_loop(0, D // 16, zcol, carry)
        lax.fori_loop(0, _CH, zrow, 0)

        rows0 = s * _RPS
        def zspm(r, carry):
            pltpu.sync_copy(mbuf, sh_m.at[pl.ds(rows0 + r * _CH, _CH)])
            return carry
        lax.fori_loop(0, _RPS // _CH, zspm, 0)
        plsc.subcore_barrier()

        pltpu.sync_copy(row_h.at[pl.ds(wid * _CPT, _CPT)], ridx)

        def step(j, carry):
            base = (wid * _CPT + j) * _CH
            pltpu.async_copy(m_h.at[pl.ds(base, _CH)], mbuf, lsem).wait()
            pltpu.async_copy(mbuf, sh_m.at[ridx.at[j]], asem, add=True).wait()
            return carry

        lax.fori_loop(0, _CPT, step, 0)
        plsc.subcore_barrier()

        @pl.when(c == 0)
        def _():
            pltpu.sync_copy(sh_m.at[pl.ds(rows0, _RPS)],
                            pm0_h.at[pl.ds(rows0, _RPS)])

        @pl.when(c == 1)
        def _():
            pltpu.sync_copy(sh_m.at[pl.ds(rows0, _RPS)],
                            pm1_h.at[pl.ds(rows0, _RPS)])

    f = pl.kernel(
        body,
        out_type=[
            jax.ShapeDtypeStruct((_NP, D), _f32),
            jax.ShapeDtypeStruct((_NP, D), _f32),
        ],
        mesh=_mesh(),
        scratch_types=[
            pltpu.VMEM((_CPT, _CH), jnp.int32),
            pltpu.VMEM((_CH, D), _f32),
            pltpu.VMEM_SHARED((_NP, D), _f32),
            pltpu.SemaphoreType.DMA,
            pltpu.SemaphoreType.DMA,
        ],
    )
    return f(m, row2d)


def _sc_scatter_t(t16, row2d):
    def body(t_h, row_h, pt0_h, pt1_h,
             ridx, tbuf, sh_t, lsem, asem):
        c = lax.axis_index("c")
        s = lax.axis_index("s")
        wid = s * _NCORES + c

        def zrow(i, carry):
            tbuf[i, pl.ds(0, 16)] = jnp.zeros((16,), _f32)
            return carry
        lax.fori_loop(0, _CH, zrow, 0)

        rows0 = s * _RPS
        def zspm(r, carry):
            pltpu.sync_copy(tbuf, sh_t.at[pl.ds(rows0 + r * _CH, _CH)])
            return carry
        lax.fori_loop(0, _RPS // _CH, zspm, 0)
        plsc.subcore_barrier()

        pltpu.sync_copy(row_h.at[pl.ds(wid * _CPT, _CPT)], ridx)

        def step(j, carry):
            base = (wid * _CPT + j) * _CH
            pltpu.async_copy(t_h.at[pl.ds(base, _CH)], tbuf, lsem).wait()
            pltpu.async_copy(tbuf, sh_t.at[ridx.at[j]], asem, add=True).wait()
            return carry

        lax.fori_loop(0, _CPT, step, 0)
        plsc.subcore_barrier()

        @pl.when(c == 0)
        def _():
            pltpu.sync_copy(sh_t.at[pl.ds(rows0, _RPS)],
                            pt0_h.at[pl.ds(rows0, _RPS)])

        @pl.when(c == 1)
        def _():
            pltpu.sync_copy(sh_t.at[pl.ds(rows0, _RPS)],
                            pt1_h.at[pl.ds(rows0, _RPS)])

    f = pl.kernel(
        body,
        out_type=[
            jax.ShapeDtypeStruct((_NP, 16), _f32),
            jax.ShapeDtypeStruct((_NP, 16), _f32),
        ],
        mesh=_mesh(),
        scratch_types=[
            pltpu.VMEM((_CPT, _CH), jnp.int32),
            pltpu.VMEM((_CH, 16), _f32),
            pltpu.VMEM_SHARED((_NP, 16), _f32),
            pltpu.SemaphoreType.DMA,
            pltpu.SemaphoreType.DMA,
        ],
        compiler_params=_SC_LINEAR,
    )
    return f(t16, row2d)


def _tc1(cr, cc, usel, vsel):
    grid = _EP // _BE1

    def body(cr_ref, cc_ref, u_ref, v_ref, cd_ref, rad_ref, ss_ref):
        cd = cr_ref[...] - cc_ref[...]
        cd_ref[...] = cd
        u = u_ref[...]
        v = v_ref[...]
        r = (jnp.dot(cd, u[0], preferred_element_type=_f32)
             * jnp.dot(cd, v[0], preferred_element_type=_f32)
             + jnp.dot(cd, u[1], preferred_element_type=_f32)
             * jnp.dot(cd, v[1], preferred_element_type=_f32)
             + jnp.dot(cd, u[2], preferred_element_type=_f32)
             * jnp.dot(cd, v[2], preferred_element_type=_f32))
        rad_ref[...] = r
        part = jnp.sum(r * r, axis=0, keepdims=True)

        @pl.when(pl.program_id(0) == 0)
        def _():
            ss_ref[...] = jnp.zeros_like(ss_ref)

        ss_ref[...] += jnp.broadcast_to(part, ss_ref.shape)

    return pl.pallas_call(
        body,
        grid=(grid,),
        in_specs=[
            pl.BlockSpec((_BE1, 16), lambda i: (i, 0)),
            pl.BlockSpec((_BE1, 16), lambda i: (i, 0)),
            pl.BlockSpec((3, 16, 16), lambda i: (0, 0, 0)),
            pl.BlockSpec((3, 16, 16), lambda i: (0, 0, 0)),
        ],
        out_specs=[
            pl.BlockSpec((_BE1, 16), lambda i: (i, 0)),
            pl.BlockSpec((_BE1, 16), lambda i: (i, 0)),
            pl.BlockSpec((8, 16), lambda i: (0, 0)),
        ],
        out_shape=[
            jax.ShapeDtypeStruct((_EP, 16), _f32),
            jax.ShapeDtypeStruct((_EP, 16), _f32),
            jax.ShapeDtypeStruct((8, 16), _f32),
        ],
        compiler_params=pltpu.CompilerParams(
            dimension_semantics=("arbitrary",)),
    )(cr, cc, usel, vsel)


def _tc2(rad, cd, xs, xt, ea, ss, weights):
    grid = _EP // _BE2

    def body(rad_ref, cd_ref, xs_ref, xt_ref, ea_ref, ss_ref,
             pe_w1t_r, pe_b1_r, pe_w2t_r, pe_b2_r, sel_r,
             pw1_r, pb1_r, pw2_r, pb2_r,
             niA_r, niB_r, nib_r,
             mmWni_r, mmWpos_r, mmWea_r, mmb1_r, mmw2t_r, mmb2_r,
             cmw1t_r, cmb1_r, cmw2t_r, expand_r,
             m_ref, t_ref):
        ss = ss_ref[0:1, :]
        rinv = 1.0 / jnp.maximum(jnp.sqrt(ss), 1e-12)
        rflat = rad_ref[...] * rinv
        h1 = _silu(jnp.dot(rflat, pe_w1t_r[...], preferred_element_type=_f32)
                   + pe_b1_r[...])
        cdiff = jnp.dot(h1, pe_w2t_r[...], preferred_element_type=_f32) + pe_b2_r[...]
        dist = jnp.sqrt(jnp.sum(cdiff * cdiff, axis=1, keepdims=True))
        dirv = cdiff / (dist + 1e-8)
        scaled = jnp.dot(cdiff, sel_r[...], preferred_element_type=_f32)
        dd = jnp.concatenate(
            [dist, dirv, jnp.zeros_like(dirv[:, :7])], axis=1)
        enc = jnp.concatenate([jnp.sin(scaled), jnp.cos(scaled), dd], axis=1)
        pos = (jnp.dot(_silu(jnp.dot(enc, pw1_r[...], preferred_element_type=_f32)
                             + pb1_r[...]),
                       pw2_r[...], preferred_element_type=_f32) + pb2_r[...])
        ni = _silu(jnp.dot(xs_ref[...], niA_r[...], preferred_element_type=_f32)
                   + jnp.dot(xt_ref[...], niB_r[...], preferred_element_type=_f32)
                   + nib_r[...])
        m1 = _silu(jnp.dot(ni, mmWni_r[...], preferred_element_type=_f32)
                   + jnp.dot(pos, mmWpos_r[...], preferred_element_type=_f32)
                   + jnp.dot(ea_ref[...], mmWea_r[...], preferred_element_type=_f32)
                   + mmb1_r[...])
        m = _silu(jnp.dot(m1, mmw2t_r[...], preferred_element_type=_f32)
                  + mmb2_r[...])
        m_ref[...] = m
        sc = jnp.dot(_silu(jnp.dot(m, cmw1t_r[...], preferred_element_type=_f32)
                           + cmb1_r[...]),
                     cmw2t_r[...], preferred_element_type=_f32)
        sexp = jnp.dot(sc, expand_r[...], preferred_element_type=_f32)
        cnt_lane = (lax.broadcasted_iota(jnp.int32, (1, 16), 1) == 12
                    ).astype(_f32)
        t_ref[...] = cd_ref[...] * sexp + cnt_lane

    full = lambda shp: pl.BlockSpec(shp, lambda i: tuple(0 for _ in shp))
    w_specs = [full(w.shape) for w in weights]
    return pl.pallas_call(
        body,
        grid=(grid,),
        in_specs=[
            pl.BlockSpec((_BE2, 16), lambda i: (i, 0)),
            pl.BlockSpec((_BE2, 16), lambda i: (i, 0)),
            pl.BlockSpec((_BE2, D), lambda i: (i, 0)),
            pl.BlockSpec((_BE2, D), lambda i: (i, 0)),
            pl.BlockSpec((_BE2, 16), lambda i: (i, 0)),
            pl.BlockSpec((8, 16), lambda i: (0, 0)),
        ] + w_specs,
        out_specs=[
            pl.BlockSpec((_BE2, D), lambda i: (i, 0)),
            pl.BlockSpec((_BE2, 16), lambda i: (i, 0)),
        ],
        out_shape=[
            jax.ShapeDtypeStruct((_EP, D), _f32),
            jax.ShapeDtypeStruct((_EP, 16), _f32),
        ],
        compiler_params=pltpu.CompilerParams(
            dimension_semantics=("arbitrary",)),
    )(rad, cd, xs, xt, ea, ss, *weights)


def _tc3(pm0, pm1, pt0, pt1, xp, coordp, geo_wt, nmA, nmB, nmb1, nmw2t, nmb2):
    grid = _NP // _BN

    def body(pm0_r, pm1_r, pt0_r, pt1_r, x_r, cp_r,
             geo_r, nmA_r, nmB_r, nmb1_r, nmw2t_r, nmb2_r,
             h_ref, co_ref):
        aggm = pm0_r[...] + pm1_r[...]
        aggt = pt0_r[...] + pt1_r[...]
        agg = jnp.dot(aggm, geo_r[...], preferred_element_type=_f32)
        xv = x_r[...]
        h = (jnp.dot(_silu(jnp.dot(xv, nmA_r[...], preferred_element_type=_f32)
                           + jnp.dot(agg, nmB_r[...], preferred_element_type=_f32)
                           + nmb1_r[...]),
                     nmw2t_r[...], preferred_element_type=_f32)
             + nmb2_r[...] + xv)
        h_ref[...] = h
        cnt = jnp.maximum(aggt[:, 12:13], 1.0)
        co_ref[...] = cp_r[...] + aggt / cnt

    full = lambda shp: pl.BlockSpec(shp, lambda i: tuple(0 for _ in shp))
    return pl.pallas_call(
        body,
        grid=(grid,),
        in_specs=[
            pl.BlockSpec((_BN, D), lambda i: (i, 0)),
            pl.BlockSpec((_BN, D), lambda i: (i, 0)),
            pl.BlockSpec((_BN, 16), lambda i: (i, 0)),
            pl.BlockSpec((_BN, 16), lambda i: (i, 0)),
            pl.BlockSpec((_BN, D), lambda i: (i, 0)),
            pl.BlockSpec((_BN, 16), lambda i: (i, 0)),
            full((D, D)), full((D, D)), full((D, D)),
            full((1, D)), full((D, D)), full((1, D)),
        ],
        out_specs=[
            pl.BlockSpec((_BN, D), lambda i: (i, 0)),
            pl.BlockSpec((_BN, 16), lambda i: (i, 0)),
        ],
        out_shape=[
            jax.ShapeDtypeStruct((_NP, D), _f32),
            jax.ShapeDtypeStruct((_NP, 16), _f32),
        ],
        compiler_params=pltpu.CompilerParams(
            dimension_semantics=("arbitrary",)),
    )(pm0, pm1, pt0, pt1, xp, coordp, geo_wt, nmA, nmB, nmb1, nmw2t, nmb2)


def _radial_selectors():
    u = np.zeros((3, 16, 16), np.float32)
    v = np.zeros((3, 16, 16), np.float32)
    for j in range(3):
        for i in range(4):
            for k in range(4):
                u[j, 3 * i + j, 4 * i + k] = 1.0
                v[j, 3 * k + j, 4 * i + k] = 1.0
    return jnp.asarray(u), jnp.asarray(v)


def _expand_mat():
    ex = np.zeros((16, 16), np.float32)
    for i in range(4):
        for j in range(3):
            ex[i, 3 * i + j] = 1.0
    return jnp.asarray(ex)


def _sc_gather(xp, coordp, row2d, col2d):
    xs, xt = _sc_gather_x(xp, row2d, col2d)
    cr, cc = _sc_gather_coord(coordp, row2d, col2d)
    return xs, xt, cr, cc


def _sc_scatter(m, t16, row2d):
    pm0, pm1 = _sc_scatter_m(m, row2d)
    pt0, pt1 = _sc_scatter_t(t16, row2d)
    return pm0, pm1, pt0, pt1


def kernel(x, coord, edge_attr, edge_index, pe_w1, pe_b1, pe_w2, pe_b2,
           pe_p_w1, pe_p_b1, pe_p_w2, pe_p_b2, ni_w, ni_b, mm_w1, mm_b1,
           mm_w2, mm_b2, geo_w, nm_w1, nm_b1, nm_w2, nm_b2, cm_w1, cm_b1,
           cm_w2, frequencies):
    n = x.shape[0]
    e = edge_index.shape[1]

    xp = jnp.zeros((_NP, D), _f32).at[:n].set(x)
    coordp = jnp.zeros((_NP, 16), _f32).at[:n, :12].set(coord.reshape(n, 12))
    row = jnp.full((_EP,), n, jnp.int32).at[:e].set(edge_index[0])
    col = jnp.full((_EP,), n, jnp.int32).at[:e].set(edge_index[1])
    row2d = row.reshape(_EP // _CH, _CH)
    col2d = col.reshape(_EP // _CH, _CH)
    eap = jnp.zeros((_EP, 16), _f32).at[:e].set(edge_attr)

    usel, vsel = _radial_selectors()

    pe_w1t = pe_w1.T
    pe_b1r = pe_b1.reshape(1, D)
    pe_w2t = jnp.zeros((D, 8), _f32).at[:, :3].set(pe_w2.T)
    pe_b2r = jnp.zeros((1, 8), _f32).at[0, :3].set(pe_b2)
    sel = jnp.zeros((8, 128), _f32)
    for i in range(3):
        sel = sel.at[i, i * NFB:(i + 1) * NFB].set(frequencies)
    pw1 = jnp.zeros((272, 32), _f32)
    pw1 = pw1.at[0:96].set(pe_p_w1[:, 0:96].T)
    pw1 = pw1.at[128:224].set(pe_p_w1[:, 96:192].T)
    pw1 = pw1.at[256].set(pe_p_w1[:, 192])
    pw1 = pw1.at[257:260].set(pe_p_w1[:, 193:196].T)
    pb1 = pe_p_b1.reshape(1, 32)
    pw2 = pe_p_w2.T
    pb2 = pe_p_b2.reshape(1, 32)
    niA = ni_w[:, :D].T
    niB = ni_w[:, D:].T
    nib = ni_b.reshape(1, D)
    mmWni = mm_w1[:, :D].T
    mmWpos = mm_w1[:, D:D + 32].T
    mmWea = mm_w1[:, D + 32:].T
    mmb1 = mm_b1.reshape(1, D)
    mmw2t = mm_w2.T
    mmb2 = mm_b2.reshape(1, D)
    cmw1t = cm_w1.T
    cmb1 = cm_b1.reshape(1, D)
    cmw2t = jnp.zeros((D, 16), _f32).at[:, :4].set(cm_w2.T)
    expand = _expand_mat()
    geo_wt = geo_w.T
    nmA = nm_w1[:, :D].T
    nmB = nm_w1[:, D:].T
    nmb1 = nm_b1.reshape(1, D)
    nmw2t = nm_w2.T
    nmb2 = nm_b2.reshape(1, D)

    xs, xt, cr, cc = _sc_gather(xp, coordp, row2d, col2d)
    cd, rad, ss = _tc1(cr, cc, usel, vsel)
    weights = [pe_w1t, pe_b1r, pe_w2t, pe_b2r, sel,
               pw1, pb1, pw2, pb2,
               niA, niB, nib,
               mmWni, mmWpos, mmWea, mmb1, mmw2t, mmb2,
               cmw1t, cmb1, cmw2t, expand]
    m, t16 = _tc2(rad, cd, xs, xt, eap, ss, weights)
    pm0, pm1, pt0, pt1 = _sc_scatter(m, t16, row2d)
    h_out, co = _tc3(pm0, pm1, pt0, pt1, xp, coordp,
                     geo_wt, nmA, nmB, nmb1, nmw2t, nmb2)

    return h_out[:n], co[:n, :12].reshape(n, 4, 3)


# speedup vs baseline: 22.7668x; 1.0647x over previous
import jax
import jax.numpy as jnp
import numpy as np
from jax import lax
from jax.experimental import pallas as pl
from jax.experimental.pallas import tpu as pltpu
from jax.experimental.pallas import tpu_sc as plsc

N = 10000
E = 160000
D = 128
NFB = 32

_NCORES = 2
_NSUB = 16
_NTILES = _NCORES * _NSUB
_CH = 128
_CPT = 40
_EP = _NTILES * _CPT * _CH
_NP = 10240
_RPS = _NP // _NSUB

_BE1 = 4096
_BE2 = 2048
_BN = 2048

_f32 = jnp.float32

_SC_LINEAR = pltpu.CompilerParams(use_tc_tiling_on_sc=False)


def _silu(v):
    return v * jax.nn.sigmoid(v)


def _mesh():
    return plsc.VectorSubcoreMesh(core_axis_name="c", subcore_axis_name="s")


_NB = 4
_NU = 2 * _CPT


def _gather_kernel(width, linear):
    def body(tab_h, row_h, col_h, os_h, ot_h,
             ridx, cidx, b0, b1, b2, b3,
             g0, g1, g2, g3, s0, s1, s2, s3):
        bufs = (b0, b1, b2, b3)
        gsems = (g0, g1, g2, g3)
        ssems = (s0, s1, s2, s3)
        c = lax.axis_index("c")
        s = lax.axis_index("s")
        wid = s * _NCORES + c
        base0 = wid * _CPT
        pltpu.sync_copy(row_h.at[pl.ds(base0, _CPT)], ridx)
        pltpu.sync_copy(col_h.at[pl.ds(base0, _CPT)], cidx)

        def issue_g(j, kind, b):
            idx = ridx if kind == 0 else cidx
            pltpu.async_copy(tab_h.at[idx.at[j]], bufs[b], gsems[b])

        def wait_g(b):
            pltpu.make_async_copy(tab_h.at[pl.ds(0, _CH)], bufs[b],
                                  gsems[b]).wait()

        def issue_s(j, kind, b):
            out = os_h if kind == 0 else ot_h
            pltpu.async_copy(bufs[b], out.at[pl.ds((base0 + j) * _CH, _CH)],
                             ssems[b])

        def wait_s(b):
            pltpu.make_async_copy(bufs[b], os_h.at[pl.ds(0, _CH)],
                                  ssems[b]).wait()

        issue_g(0, 0, 0)
        issue_g(0, 1, 1)
        issue_g(1, 0, 2)

        def outer(g, carry):
            u0 = g * (2 * _NB)
            for ul in range(2 * _NB):
                b = ul % _NB
                kind = ul % 2
                j = g * _NB + ul // 2
                u = u0 + ul
                wait_g(b)
                issue_s(j, kind, b)

                @pl.when(u >= 1)
                def _():
                    wait_s((b + _NB - 1) % _NB)

                @pl.when(u + 3 < _NU)
                def _():
                    kind3 = (ul + 3) % 2
                    j3 = g * _NB + (ul + 3) // 2 if ul + 3 < 2 * _NB \
                        else (g + 1) * _NB + (ul + 3 - 2 * _NB) // 2
                    issue_g(j3, kind3, (b + 3) % _NB)
            return carry

        lax.fori_loop(0, _NU // (2 * _NB), outer, 0)
        wait_s((_NU - 1) % _NB)

    scratch = (
        [pltpu.VMEM((_CPT, _CH), jnp.int32)] * 2
        + [pltpu.VMEM((_CH, width), _f32)] * _NB
        + [pltpu.SemaphoreType.DMA] * (2 * _NB)
    )
    kw = {}
    if linear:
        kw["compiler_params"] = _SC_LINEAR

    def run(table, row2d, col2d):
        f = pl.kernel(
            body,
            out_type=[
                jax.ShapeDtypeStruct((_EP, width), _f32),
                jax.ShapeDtypeStruct((_EP, width), _f32),
            ],
            mesh=_mesh(),
            scratch_types=scratch,
            **kw,
        )
        return f(table, row2d, col2d)

    return run


def _sc_gather_x(xp, row2d, col2d):
    return _gather_kernel(D, False)(xp, row2d, col2d)


def _sc_gather_coord(coordp, row2d, col2d):
    return _gather_kernel(16, True)(coordp, row2d, col2d)


def _scatter_kernel(width, linear, nb):
    def body(v_h, row_h, p0_h, p1_h, ridx, *rest):
        bufs = rest[:nb]
        sh = rest[nb]
        lsems = rest[nb + 1:2 * nb + 1]
        asems = rest[2 * nb + 1:]
        c = lax.axis_index("c")
        s = lax.axis_index("s")
        wid = s * _NCORES + c
        base0 = wid * _CPT

        pltpu.sync_copy(row_h.at[pl.ds(base0, _CPT)], ridx)

        zb = bufs[0]
        def zrow(i, carry):
            def zcol(k, carry2):
                zb[i, pl.ds(k * 16, 16)] = jnp.zeros((16,), _f32)
                return carry2
            return lax.fori_loop(0, width // 16, zcol, carry)
        lax.fori_loop(0, _CH, zrow, 0)

        rows0 = s * _RPS
        def zspm(r, carry):
            pltpu.sync_copy(zb, sh.at[pl.ds(rows0 + r * _CH, _CH)])
            return carry
        lax.fori_loop(0, _RPS // _CH, zspm, 0)
        plsc.subcore_barrier()

        def issue_l(j, b):
            pltpu.async_copy(v_h.at[pl.ds((base0 + j) * _CH, _CH)],
                             bufs[b], lsems[b])

        def wait_l(b):
            pltpu.make_async_copy(v_h.at[pl.ds(0, _CH)], bufs[b],
                                  lsems[b]).wait()

        def issue_a(j, b):
            pltpu.async_copy(bufs[b], sh.at[ridx.at[j]], asems[b], add=True)

        def wait_a(b):
            pltpu.make_async_copy(bufs[b], sh.at[pl.ds(0, _CH)],
                                  asems[b]).wait()

        for p in range(nb - 1):
            issue_l(p, p)

        def outer(g, carry):
            for b in range(nb):
                j = g * nb + b
                wait_l(b)
                issue_a(j, b)

                @pl.when(j >= 1)
                def _():
                    wait_a((b + nb - 1) % nb)

                @pl.when(j + nb - 1 < _CPT)
                def _():
                    issue_l(j + nb - 1, (b + nb - 1) % nb)
            return carry

        lax.fori_loop(0, _CPT // nb, outer, 0)
        wait_a((_CPT - 1) % nb)
        plsc.subcore_barrier()

        @pl.when(c == 0)
        def _():
            pltpu.sync_copy(sh.at[pl.ds(rows0, _RPS)],
                            p0_h.at[pl.ds(rows0, _RPS)])

        @pl.when(c == 1)
        def _():
            pltpu.sync_copy(sh.at[pl.ds(rows0, _RPS)],
                            p1_h.at[pl.ds(rows0, _RPS)])

    scratch = (
        [pltpu.VMEM((_CPT, _CH), jnp.int32)]
        + [pltpu.VMEM((_CH, width), _f32)] * nb
        + [pltpu.VMEM_SHARED((_NP, width), _f32)]
        + [pltpu.SemaphoreType.DMA] * (2 * nb)
    )
    kw = {}
    if linear:
        kw["compiler_params"] = _SC_LINEAR

    def run(vals, row2d):
        f = pl.kernel(
            body,
            out_type=[
                jax.ShapeDtypeStruct((_NP, width), _f32),
                jax.ShapeDtypeStruct((_NP, width), _f32),
            ],
            mesh=_mesh(),
            scratch_types=scratch,
            **kw,
        )
        return f(vals, row2d)

    return run


def _sc_scatter_m(m, row2d):
    return _scatter_kernel(D, False, 2)(m, row2d)


def _sc_scatter_t(t16, row2d):
    return _scatter_kernel(16, True, 4)(t16, row2d)


def _tc1(cr, cc, usel, vsel):
    grid = _EP // _BE1

    def body(cr_ref, cc_ref, u_ref, v_ref, cd_ref, rad_ref, ss_ref):
        cd = cr_ref[...] - cc_ref[...]
        cd_ref[...] = cd
        u = u_ref[...]
        v = v_ref[...]
        r = (jnp.dot(cd, u[0], preferred_element_type=_f32)
             * jnp.dot(cd, v[0], preferred_element_type=_f32)
             + jnp.dot(cd, u[1], preferred_element_type=_f32)
             * jnp.dot(cd, v[1], preferred_element_type=_f32)
             + jnp.dot(cd, u[2], preferred_element_type=_f32)
             * jnp.dot(cd, v[2], preferred_element_type=_f32))
        rad_ref[...] = r
        part = jnp.sum(r * r, axis=0, keepdims=True)

        @pl.when(pl.program_id(0) == 0)
        def _():
            ss_ref[...] = jnp.zeros_like(ss_ref)

        ss_ref[...] += jnp.broadcast_to(part, ss_ref.shape)

    return pl.pallas_call(
        body,
        grid=(grid,),
        in_specs=[
            pl.BlockSpec((_BE1, 16), lambda i: (i, 0)),
            pl.BlockSpec((_BE1, 16), lambda i: (i, 0)),
            pl.BlockSpec((3, 16, 16), lambda i: (0, 0, 0)),
            pl.BlockSpec((3, 16, 16), lambda i: (0, 0, 0)),
        ],
        out_specs=[
            pl.BlockSpec((_BE1, 16), lambda i: (i, 0)),
            pl.BlockSpec((_BE1, 16), lambda i: (i, 0)),
            pl.BlockSpec((8, 16), lambda i: (0, 0)),
        ],
        out_shape=[
            jax.ShapeDtypeStruct((_EP, 16), _f32),
            jax.ShapeDtypeStruct((_EP, 16), _f32),
            jax.ShapeDtypeStruct((8, 16), _f32),
        ],
        compiler_params=pltpu.CompilerParams(
            dimension_semantics=("arbitrary",)),
    )(cr, cc, usel, vsel)


def _tc2(rad, cd, xs, xt, ea, ss, weights):
    grid = _EP // _BE2

    def body(rad_ref, cd_ref, xs_ref, xt_ref, ea_ref, ss_ref,
             pe_w1t_r, pe_b1_r, pe_w2t_r, pe_b2_r, sel_r,
             pw1_r, pb1_r, pw2_r, pb2_r,
             niA_r, niB_r, nib_r,
             mmWni_r, mmWpos_r, mmWea_r, mmb1_r, mmw2t_r, mmb2_r,
             cmw1t_r, cmb1_r, cmw2t_r, expand_r,
             m_ref, t_ref):
        ss = ss_ref[0:1, :]
        rinv = 1.0 / jnp.maximum(jnp.sqrt(ss), 1e-12)
        rflat = rad_ref[...] * rinv
        h1 = _silu(jnp.dot(rflat, pe_w1t_r[...], preferred_element_type=_f32)
                   + pe_b1_r[...])
        cdiff = jnp.dot(h1, pe_w2t_r[...], preferred_element_type=_f32) + pe_b2_r[...]
        dist = jnp.sqrt(jnp.sum(cdiff * cdiff, axis=1, keepdims=True))
        dirv = cdiff / (dist + 1e-8)
        scaled = jnp.dot(cdiff, sel_r[...], preferred_element_type=_f32)
        dd = jnp.concatenate(
            [dist, dirv, jnp.zeros_like(dirv[:, :7])], axis=1)
        enc = jnp.concatenate([jnp.sin(scaled), jnp.cos(scaled), dd], axis=1)
        pos = (jnp.dot(_silu(jnp.dot(enc, pw1_r[...], preferred_element_type=_f32)
                             + pb1_r[...]),
                       pw2_r[...], preferred_element_type=_f32) + pb2_r[...])
        ni = _silu(jnp.dot(xs_ref[...], niA_r[...], preferred_element_type=_f32)
                   + jnp.dot(xt_ref[...], niB_r[...], preferred_element_type=_f32)
                   + nib_r[...])
        m1 = _silu(jnp.dot(ni, mmWni_r[...], preferred_element_type=_f32)
                   + jnp.dot(pos, mmWpos_r[...], preferred_element_type=_f32)
                   + jnp.dot(ea_ref[...], mmWea_r[...], preferred_element_type=_f32)
                   + mmb1_r[...])
        m = _silu(jnp.dot(m1, mmw2t_r[...], preferred_element_type=_f32)
                  + mmb2_r[...])
        m_ref[...] = m
        sc = jnp.dot(_silu(jnp.dot(m, cmw1t_r[...], preferred_element_type=_f32)
                           + cmb1_r[...]),
                     cmw2t_r[...], preferred_element_type=_f32)
        sexp = jnp.dot(sc, expand_r[...], preferred_element_type=_f32)
        cnt_lane = (lax.broadcasted_iota(jnp.int32, (1, 16), 1) == 12
                    ).astype(_f32)
        t_ref[...] = cd_ref[...] * sexp + cnt_lane

    full = lambda shp: pl.BlockSpec(shp, lambda i: tuple(0 for _ in shp))
    w_specs = [full(w.shape) for w in weights]
    return pl.pallas_call(
        body,
        grid=(grid,),
        in_specs=[
            pl.BlockSpec((_BE2, 16), lambda i: (i, 0)),
            pl.BlockSpec((_BE2, 16), lambda i: (i, 0)),
            pl.BlockSpec((_BE2, D), lambda i: (i, 0)),
            pl.BlockSpec((_BE2, D), lambda i: (i, 0)),
            pl.BlockSpec((_BE2, 16), lambda i: (i, 0)),
            pl.BlockSpec((8, 16), lambda i: (0, 0)),
        ] + w_specs,
        out_specs=[
            pl.BlockSpec((_BE2, D), lambda i: (i, 0)),
            pl.BlockSpec((_BE2, 16), lambda i: (i, 0)),
        ],
        out_shape=[
            jax.ShapeDtypeStruct((_EP, D), _f32),
            jax.ShapeDtypeStruct((_EP, 16), _f32),
        ],
        compiler_params=pltpu.CompilerParams(
            dimension_semantics=("arbitrary",)),
    )(rad, cd, xs, xt, ea, ss, *weights)


def _tc3(pm0, pm1, pt0, pt1, xp, coordp, geo_wt, nmA, nmB, nmb1, nmw2t, nmb2):
    grid = _NP // _BN

    def body(pm0_r, pm1_r, pt0_r, pt1_r, x_r, cp_r,
             geo_r, nmA_r, nmB_r, nmb1_r, nmw2t_r, nmb2_r,
             h_ref, co_ref):
        aggm = pm0_r[...] + pm1_r[...]
        aggt = pt0_r[...] + pt1_r[...]
        agg = jnp.dot(aggm, geo_r[...], preferred_element_type=_f32)
        xv = x_r[...]
        h = (jnp.dot(_silu(jnp.dot(xv, nmA_r[...], preferred_element_type=_f32)
                           + jnp.dot(agg, nmB_r[...], preferred_element_type=_f32)
                           + nmb1_r[...]),
                     nmw2t_r[...], preferred_element_type=_f32)
             + nmb2_r[...] + xv)
        h_ref[...] = h
        cnt = jnp.maximum(aggt[:, 12:13], 1.0)
        co_ref[...] = cp_r[...] + aggt / cnt

    full = lambda shp: pl.BlockSpec(shp, lambda i: tuple(0 for _ in shp))
    return pl.pallas_call(
        body,
        grid=(grid,),
        in_specs=[
            pl.BlockSpec((_BN, D), lambda i: (i, 0)),
            pl.BlockSpec((_BN, D), lambda i: (i, 0)),
            pl.BlockSpec((_BN, 16), lambda i: (i, 0)),
            pl.BlockSpec((_BN, 16), lambda i: (i, 0)),
            pl.BlockSpec((_BN, D), lambda i: (i, 0)),
            pl.BlockSpec((_BN, 16), lambda i: (i, 0)),
            full((D, D)), full((D, D)), full((D, D)),
            full((1, D)), full((D, D)), full((1, D)),
        ],
        out_specs=[
            pl.BlockSpec((_BN, D), lambda i: (i, 0)),
            pl.BlockSpec((_BN, 16), lambda i: (i, 0)),
        ],
        out_shape=[
            jax.ShapeDtypeStruct((_NP, D), _f32),
            jax.ShapeDtypeStruct((_NP, 16), _f32),
        ],
        compiler_params=pltpu.CompilerParams(
            dimension_semantics=("arbitrary",)),
    )(pm0, pm1, pt0, pt1, xp, coordp, geo_wt, nmA, nmB, nmb1, nmw2t, nmb2)


def _radial_selectors():
    u = np.zeros((3, 16, 16), np.float32)
    v = np.zeros((3, 16, 16), np.float32)
    for j in range(3):
        for i in range(4):
            for k in range(4):
                u[j, 3 * i + j, 4 * i + k] = 1.0
                v[j, 3 * k + j, 4 * i + k] = 1.0
    return jnp.asarray(u), jnp.asarray(v)


def _expand_mat():
    ex = np.zeros((16, 16), np.float32)
    for i in range(4):
        for j in range(3):
            ex[i, 3 * i + j] = 1.0
    return jnp.asarray(ex)


def _sc_gather(xp, coordp, row2d, col2d):
    xs, xt = _sc_gather_x(xp, row2d, col2d)
    cr, cc = _sc_gather_coord(coordp, row2d, col2d)
    return xs, xt, cr, cc


def _sc_scatter(m, t16, row2d):
    pm0, pm1 = _sc_scatter_m(m, row2d)
    pt0, pt1 = _sc_scatter_t(t16, row2d)
    return pm0, pm1, pt0, pt1


def kernel(x, coord, edge_attr, edge_index, pe_w1, pe_b1, pe_w2, pe_b2,
           pe_p_w1, pe_p_b1, pe_p_w2, pe_p_b2, ni_w, ni_b, mm_w1, mm_b1,
           mm_w2, mm_b2, geo_w, nm_w1, nm_b1, nm_w2, nm_b2, cm_w1, cm_b1,
           cm_w2, frequencies):
    n = x.shape[0]
    e = edge_index.shape[1]

    xp = jnp.zeros((_NP, D), _f32).at[:n].set(x)
    coordp = jnp.zeros((_NP, 16), _f32).at[:n, :12].set(coord.reshape(n, 12))
    row = jnp.full((_EP,), n, jnp.int32).at[:e].set(edge_index[0])
    col = jnp.full((_EP,), n, jnp.int32).at[:e].set(edge_index[1])
    row2d = row.reshape(_EP // _CH, _CH)
    col2d = col.reshape(_EP // _CH, _CH)
    eap = jnp.zeros((_EP, 16), _f32).at[:e].set(edge_attr)

    usel, vsel = _radial_selectors()

    pe_w1t = pe_w1.T
    pe_b1r = pe_b1.reshape(1, D)
    pe_w2t = jnp.zeros((D, 8), _f32).at[:, :3].set(pe_w2.T)
    pe_b2r = jnp.zeros((1, 8), _f32).at[0, :3].set(pe_b2)
    sel = jnp.zeros((8, 128), _f32)
    for i in range(3):
        sel = sel.at[i, i * NFB:(i + 1) * NFB].set(frequencies)
    pw1 = jnp.zeros((272, 32), _f32)
    pw1 = pw1.at[0:96].set(pe_p_w1[:, 0:96].T)
    pw1 = pw1.at[128:224].set(pe_p_w1[:, 96:192].T)
    pw1 = pw1.at[256].set(pe_p_w1[:, 192])
    pw1 = pw1.at[257:260].set(pe_p_w1[:, 193:196].T)
    pb1 = pe_p_b1.reshape(1, 32)
    pw2 = pe_p_w2.T
    pb2 = pe_p_b2.reshape(1, 32)
    niA = ni_w[:, :D].T
    niB = ni_w[:, D:].T
    nib = ni_b.reshape(1, D)
    mmWni = mm_w1[:, :D].T
    mmWpos = mm_w1[:, D:D + 32].T
    mmWea = mm_w1[:, D + 32:].T
    mmb1 = mm_b1.reshape(1, D)
    mmw2t = mm_w2.T
    mmb2 = mm_b2.reshape(1, D)
    cmw1t = cm_w1.T
    cmb1 = cm_b1.reshape(1, D)
    cmw2t = jnp.zeros((D, 16), _f32).at[:, :4].set(cm_w2.T)
    expand = _expand_mat()
    geo_wt = geo_w.T
    nmA = nm_w1[:, :D].T
    nmB = nm_w1[:, D:].T
    nmb1 = nm_b1.reshape(1, D)
    nmw2t = nm_w2.T
    nmb2 = nm_b2.reshape(1, D)

    xs, xt, cr, cc = _sc_gather(xp, coordp, row2d, col2d)
    cd, rad, ss = _tc1(cr, cc, usel, vsel)
    weights = [pe_w1t, pe_b1r, pe_w2t, pe_b2r, sel,
               pw1, pb1, pw2, pb2,
               niA, niB, nib,
               mmWni, mmWpos, mmWea, mmb1, mmw2t, mmb2,
               cmw1t, cmb1, cmw2t, expand]
    m, t16 = _tc2(rad, cd, xs, xt, eap, ss, weights)
    pm0, pm1, pt0, pt1 = _sc_scatter(m, t16, row2d)
    h_out, co = _tc3(pm0, pm1, pt0, pt1, xp, coordp,
                     geo_wt, nmA, nmB, nmb1, nmw2t, nmb2)

    return h_out[:n], co[:n, :12].reshape(n, 4, 3)
